# Initial kernel scaffold; baseline (speedup 1.0000x reference)
import jax
import jax.numpy as jnp
from jax.experimental import pallas as pl


def kernel(x, a, user_table, item_table, W1, b1, W2, b2, W3, b3):
    raise NotImplementedError("write your pallas kernel here")


# speedup vs baseline: 2.3549x; 2.3549x over previous
import functools

import jax
import jax.numpy as jnp
from jax import lax
from jax.experimental import pallas as pl
from jax.experimental.pallas import tpu as pltpu
from jax.experimental.pallas import tpu_sc as plsc

B = 16384
V = 100000
E = 128
H1 = 1024
H2 = 512

_NC = 2
_NS = 16
_NW = _NC * _NS
_BPW = B // _NW
_CH = 128
_NCH = _BPW // _CH

@functools.lru_cache(maxsize=None)
def _sc_gather_fn():
    mesh = plsc.VectorSubcoreMesh(core_axis_name="c", subcore_axis_name="s")

    @functools.partial(
        pl.kernel,
        mesh=mesh,
        out_type=(
            jax.ShapeDtypeStruct((B, E), jnp.float32),
            jax.ShapeDtypeStruct((B, E), jnp.float32),
        ),
        scratch_types=[
            pltpu.VMEM((_NCH, _CH), jnp.int32),
            pltpu.VMEM((_NCH, _CH), jnp.int32),
            pltpu.VMEM((_CH, E), jnp.float32),
            pltpu.VMEM((_CH, E), jnp.float32),
            pltpu.SemaphoreType.DMA,
            pltpu.SemaphoreType.DMA,
            pltpu.SemaphoreType.DMA,
        ],
    )
    def _sc_gather(x_hbm, a_hbm, ut_hbm, it_hbm, ue_hbm, ie_hbm,
                   xidx, aidx, buf0, buf1, gsem, psem0, psem1):
        wid = lax.axis_index("s") * _NC + lax.axis_index("c")
        base = wid * _BPW
        pltpu.sync_copy(x_hbm.at[pl.ds(wid * _NCH, _NCH)], xidx)
        pltpu.sync_copy(a_hbm.at[pl.ds(wid * _NCH, _NCH)], aidx)
        bufs = (buf0, buf1)
        psems = (psem0, psem1)
        puts = [None, None]
        for t in range(2 * _NCH):
            j = t % _NCH
            table, idxs, out = (
                (ut_hbm, xidx, ue_hbm) if t < _NCH else (it_hbm, aidx, ie_hbm))
            k = t % 2
            if puts[k] is not None:
                puts[k].wait()
            pltpu.async_copy(table.at[idxs.at[j]], bufs[k], gsem).wait()
            puts[k] = pltpu.async_copy(
                bufs[k], out.at[pl.ds(base + j * _CH, _CH)], psems[k])
        puts[0].wait()
        puts[1].wait()

    return _sc_gather


def _mlp_body(ue_ref, ie_ref, w1u_ref, w1i_ref, b1_ref, w2_ref, b2_ref,
              w3_ref, b3_ref, out_ref):
    h = jnp.dot(ue_ref[...], w1u_ref[...], preferred_element_type=jnp.float32)
    h = h + jnp.dot(ie_ref[...], w1i_ref[...], preferred_element_type=jnp.float32)
    h = h + b1_ref[...]
    h = jnp.where(h > 0, h, (jnp.exp(h) - 1.0))
    h = jnp.dot(h, w2_ref[...], preferred_element_type=jnp.float32) + b2_ref[...]
    h = jnp.where(h > 0, h, (jnp.exp(h) - 1.0))
    out_ref[...] = jnp.dot(h, w3_ref[...], preferred_element_type=jnp.float32) + b3_ref[...]


_BT = 2048


def _mlp(ue, ie, W1u, W1i, b1, W2, b2, W3, b3):
    grid = (B // _BT,)
    full = lambda i: (0, 0)
    return pl.pallas_call(
        _mlp_body,
        grid=grid,
        in_specs=[
            pl.BlockSpec((_BT, E), lambda i: (i, 0)),
            pl.BlockSpec((_BT, E), lambda i: (i, 0)),
            pl.BlockSpec((E, H1), full),
            pl.BlockSpec((E, H1), full),
            pl.BlockSpec((1, H1), full),
            pl.BlockSpec((H1, H2), full),
            pl.BlockSpec((1, H2), full),
            pl.BlockSpec((H2, 1), full),
            pl.BlockSpec((1, 1), full),
        ],
        out_specs=pl.BlockSpec((_BT, 1), lambda i: (i, 0)),
        out_shape=jax.ShapeDtypeStruct((B, 1), jnp.float32),
        compiler_params=pltpu.CompilerParams(
            dimension_semantics=("arbitrary",),
        ),
    )(ue, ie, W1u, W1i, b1, W2, b2, W3, b3)


def kernel(x, a, user_table, item_table, W1, b1, W2, b2, W3, b3):
    x2 = x.astype(jnp.int32).reshape(B // _CH, _CH)
    a2 = a.astype(jnp.int32).reshape(B // _CH, _CH)
    ue, ie = _sc_gather_fn()(x2, a2, user_table, item_table)
    return _mlp(ue, ie, W1[:E], W1[E:], b1.reshape(1, H1), W2,
                b2.reshape(1, H2), W3, b3.reshape(1, 1))


# speedup vs baseline: 2.3689x; 1.0059x over previous
import functools

import jax
import jax.numpy as jnp
from jax import lax
from jax.experimental import pallas as pl
from jax.experimental.pallas import tpu as pltpu
from jax.experimental.pallas import tpu_sc as plsc

B = 16384
V = 100000
E = 128
H1 = 1024
H2 = 512

_NC = 2
_NS = 16
_NW = _NC * _NS
_BPW = B // _NW
_CH = 128
_NCH = _BPW // _CH

@functools.lru_cache(maxsize=None)
def _sc_gather_fn():
    mesh = plsc.VectorSubcoreMesh(core_axis_name="c", subcore_axis_name="s")

    @functools.partial(
        pl.kernel,
        mesh=mesh,
        out_type=(
            jax.ShapeDtypeStruct((B, E), jnp.float32),
            jax.ShapeDtypeStruct((B, E), jnp.float32),
        ),
        scratch_types=[
            pltpu.VMEM((_NCH, _CH), jnp.int32),
            pltpu.VMEM((_NCH, _CH), jnp.int32),
            pltpu.VMEM((_CH, E), jnp.float32),
            pltpu.VMEM((_CH, E), jnp.float32),
            pltpu.SemaphoreType.DMA,
            pltpu.SemaphoreType.DMA,
            pltpu.SemaphoreType.DMA,
        ],
    )
    def _sc_gather(x_hbm, a_hbm, ut_hbm, it_hbm, ue_hbm, ie_hbm,
                   xidx, aidx, buf0, buf1, gsem, psem0, psem1):
        wid = lax.axis_index("s") * _NC + lax.axis_index("c")
        base = wid * _BPW
        pltpu.sync_copy(x_hbm.at[pl.ds(wid * _NCH, _NCH)], xidx)
        pltpu.sync_copy(a_hbm.at[pl.ds(wid * _NCH, _NCH)], aidx)
        bufs = (buf0, buf1)
        psems = (psem0, psem1)
        puts = [None, None]
        for t in range(2 * _NCH):
            j = t % _NCH
            table, idxs, out = (
                (ut_hbm, xidx, ue_hbm) if t < _NCH else (it_hbm, aidx, ie_hbm))
            k = t % 2
            if puts[k] is not None:
                puts[k].wait()
            pltpu.async_copy(table.at[idxs.at[j]], bufs[k], gsem).wait()
            puts[k] = pltpu.async_copy(
                bufs[k], out.at[pl.ds(base + j * _CH, _CH)], psems[k])
        puts[0].wait()
        puts[1].wait()

    return _sc_gather


def _mlp_body(ue_ref, ie_ref, w1_ref, b1_ref, w2_ref, b2_ref,
              w3_ref, b3_ref, out_ref):
    bf = jnp.bfloat16
    h = jnp.dot(ue_ref[...].astype(bf), w1_ref[:E],
                preferred_element_type=jnp.float32)
    h = h + jnp.dot(ie_ref[...].astype(bf), w1_ref[E:],
                    preferred_element_type=jnp.float32)
    h = h + b1_ref[...]
    h = jnp.where(h > 0, h, (jnp.exp(h) - 1.0))
    h = jnp.dot(h.astype(bf), w2_ref[...],
                preferred_element_type=jnp.float32) + b2_ref[...]
    h = jnp.where(h > 0, h, (jnp.exp(h) - 1.0))
    out_ref[...] = jnp.dot(h.astype(bf), w3_ref[...],
                           preferred_element_type=jnp.float32) + b3_ref[...]


_BT = 2048


def _mlp(ue, ie, W1, b1, W2, b2, W3, b3):
    grid = (B // _BT,)
    full = lambda i: (0, 0)
    return pl.pallas_call(
        _mlp_body,
        grid=grid,
        in_specs=[
            pl.BlockSpec((_BT, E), lambda i: (i, 0)),
            pl.BlockSpec((_BT, E), lambda i: (i, 0)),
            pl.BlockSpec((2 * E, H1), full),
            pl.BlockSpec((1, H1), full),
            pl.BlockSpec((H1, H2), full),
            pl.BlockSpec((1, H2), full),
            pl.BlockSpec((H2, 1), full),
            pl.BlockSpec((1, 1), full),
        ],
        out_specs=pl.BlockSpec((_BT, 1), lambda i: (i, 0)),
        out_shape=jax.ShapeDtypeStruct((B, 1), jnp.float32),
        compiler_params=pltpu.CompilerParams(
            dimension_semantics=("arbitrary",),
        ),
    )(ue, ie, W1, b1, W2, b2, W3, b3)


def kernel(x, a, user_table, item_table, W1, b1, W2, b2, W3, b3):
    x2 = x.astype(jnp.int32).reshape(B // _CH, _CH)
    a2 = a.astype(jnp.int32).reshape(B // _CH, _CH)
    ue, ie = _sc_gather_fn()(x2, a2, user_table, item_table)
    return _mlp(ue, ie, W1.astype(jnp.bfloat16), b1.reshape(1, H1),
                W2.astype(jnp.bfloat16), b2.reshape(1, H2),
                W3.astype(jnp.bfloat16), b3.reshape(1, 1))


# speedup vs baseline: 2.5258x; 1.0662x over previous
import functools

import jax
import jax.numpy as jnp
from jax import lax
from jax.experimental import pallas as pl
from jax.experimental.pallas import tpu as pltpu
from jax.experimental.pallas import tpu_sc as plsc

B = 16384
V = 100000
E = 128
H1 = 1024
H2 = 512

_NC = 2
_NS = 16
_NW = _NC * _NS
_BPW = B // _NW
_CH = 128
_NCH = _BPW // _CH

@functools.lru_cache(maxsize=None)
def _sc_gather_fn():
    mesh = plsc.VectorSubcoreMesh(core_axis_name="c", subcore_axis_name="s")

    @functools.partial(
        pl.kernel,
        mesh=mesh,
        out_type=(
            jax.ShapeDtypeStruct((B, E), jnp.float32),
            jax.ShapeDtypeStruct((B, E), jnp.float32),
        ),
        scratch_types=[
            pltpu.VMEM((_NCH, _CH), jnp.int32),
            pltpu.VMEM((_NCH, _CH), jnp.int32),
            pltpu.VMEM((_CH, E), jnp.float32),
            pltpu.VMEM((_CH, E), jnp.float32),
            pltpu.SemaphoreType.DMA,
            pltpu.SemaphoreType.DMA,
            pltpu.SemaphoreType.DMA,
        ],
    )
    def _sc_gather(x_hbm, a_hbm, ut_hbm, it_hbm, ue_hbm, ie_hbm,
                   xidx, aidx, buf0, buf1, gsem, psem0, psem1):
        wid = lax.axis_index("s") * _NC + lax.axis_index("c")
        base = wid * _BPW
        pltpu.sync_copy(x_hbm.at[pl.ds(wid * _NCH, _NCH)], xidx)
        pltpu.sync_copy(a_hbm.at[pl.ds(wid * _NCH, _NCH)], aidx)
        bufs = (buf0, buf1)
        psems = (psem0, psem1)
        puts = [None, None]
        for t in range(2 * _NCH):
            j = t % _NCH
            table, idxs, out = (
                (ut_hbm, xidx, ue_hbm) if t < _NCH else (it_hbm, aidx, ie_hbm))
            k = t % 2
            if puts[k] is not None:
                puts[k].wait()
            pltpu.async_copy(table.at[idxs.at[j]], bufs[k], gsem).wait()
            puts[k] = pltpu.async_copy(
                bufs[k], out.at[pl.ds(base + j * _CH, _CH)], psems[k])
        puts[0].wait()
        puts[1].wait()

    return _sc_gather


def _mlp_body(ue_ref, ie_ref, w1_ref, b1_ref, w2_ref, b2_ref,
              w3_ref, b3_ref, out_ref):
    bf = jnp.bfloat16
    h = jnp.dot(ue_ref[...].astype(bf), w1_ref[:E],
                preferred_element_type=jnp.float32)
    h = h + jnp.dot(ie_ref[...].astype(bf), w1_ref[E:],
                    preferred_element_type=jnp.float32)
    h = h + b1_ref[...]
    h = jnp.where(h > 0, h, (jnp.exp(h) - 1.0))
    h = jnp.dot(h.astype(bf), w2_ref[...],
                preferred_element_type=jnp.float32) + b2_ref[...]
    h = jnp.where(h > 0, h, (jnp.exp(h) - 1.0))
    o = jnp.dot(h.astype(bf), w3_ref[...],
                preferred_element_type=jnp.float32) + b3_ref[...]
    out_ref[...] = o.reshape(_BT // 128, 128)


_BT = 2048


def _mlp(ue, ie, W1, b1, W2, b2, W3, b3):
    grid = (B // _BT,)
    full = lambda i: (0, 0)
    return pl.pallas_call(
        _mlp_body,
        grid=grid,
        in_specs=[
            pl.BlockSpec((_BT, E), lambda i: (i, 0)),
            pl.BlockSpec((_BT, E), lambda i: (i, 0)),
            pl.BlockSpec((2 * E, H1), full),
            pl.BlockSpec((1, H1), full),
            pl.BlockSpec((H1, H2), full),
            pl.BlockSpec((1, H2), full),
            pl.BlockSpec((H2, 1), full),
            pl.BlockSpec((1, 1), full),
        ],
        out_specs=pl.BlockSpec((_BT // 128, 128), lambda i: (i, 0)),
        out_shape=jax.ShapeDtypeStruct((B // 128, 128), jnp.float32),
        compiler_params=pltpu.CompilerParams(
            dimension_semantics=("arbitrary",),
        ),
    )(ue, ie, W1, b1, W2, b2, W3, b3)


def kernel(x, a, user_table, item_table, W1, b1, W2, b2, W3, b3):
    x2 = x.astype(jnp.int32).reshape(B // _CH, _CH)
    a2 = a.astype(jnp.int32).reshape(B // _CH, _CH)
    ue, ie = _sc_gather_fn()(x2, a2, user_table, item_table)
    out = _mlp(ue, ie, W1.astype(jnp.bfloat16), b1.reshape(1, H1),
               W2.astype(jnp.bfloat16), b2.reshape(1, H2),
               W3.astype(jnp.bfloat16), b3.reshape(1, 1))
    return out.reshape(B, 1)


# speedup vs baseline: 2.6322x; 1.0421x over previous
import functools

import jax
import jax.numpy as jnp
from jax import lax
from jax.experimental import pallas as pl
from jax.experimental.pallas import tpu as pltpu
from jax.experimental.pallas import tpu_sc as plsc

B = 16384
V = 100000
E = 128
H1 = 1024
H2 = 512

_NSPLIT = 2
_BS = B // _NSPLIT

_NC = 2
_NS = 16
_NW = _NC * _NS
_BPW = _BS // _NW
_CH = 128
_NCH = _BPW // _CH


@functools.lru_cache(maxsize=None)
def _sc_gather_fn():
    mesh = plsc.VectorSubcoreMesh(core_axis_name="c", subcore_axis_name="s")

    @functools.partial(
        pl.kernel,
        mesh=mesh,
        out_type=(
            jax.ShapeDtypeStruct((_BS, E), jnp.float32),
            jax.ShapeDtypeStruct((_BS, E), jnp.float32),
        ),
        scratch_types=[
            pltpu.VMEM((_NCH, _CH), jnp.int32),
            pltpu.VMEM((_NCH, _CH), jnp.int32),
            pltpu.VMEM((_CH, E), jnp.float32),
            pltpu.VMEM((_CH, E), jnp.float32),
            pltpu.SemaphoreType.DMA,
            pltpu.SemaphoreType.DMA,
            pltpu.SemaphoreType.DMA,
        ],
    )
    def _sc_gather(x_hbm, a_hbm, ut_hbm, it_hbm, ue_hbm, ie_hbm,
                   xidx, aidx, buf0, buf1, gsem, psem0, psem1):
        wid = lax.axis_index("s") * _NC + lax.axis_index("c")
        base = wid * _BPW
        pltpu.sync_copy(x_hbm.at[pl.ds(wid * _NCH, _NCH)], xidx)
        pltpu.sync_copy(a_hbm.at[pl.ds(wid * _NCH, _NCH)], aidx)
        bufs = (buf0, buf1)
        psems = (psem0, psem1)
        puts = [None, None]
        for t in range(2 * _NCH):
            j = t % _NCH
            table, idxs, out = (
                (ut_hbm, xidx, ue_hbm) if t < _NCH else (it_hbm, aidx, ie_hbm))
            k = t % 2
            if puts[k] is not None:
                puts[k].wait()
            pltpu.async_copy(table.at[idxs.at[j]], bufs[k], gsem).wait()
            puts[k] = pltpu.async_copy(
                bufs[k], out.at[pl.ds(base + j * _CH, _CH)], psems[k])
        puts[0].wait()
        puts[1].wait()

    return _sc_gather


def _mlp_body(ue_ref, ie_ref, w1_ref, b1_ref, w2_ref, b2_ref,
              w3_ref, b3_ref, out_ref):
    bf = jnp.bfloat16
    h = jnp.dot(ue_ref[...].astype(bf), w1_ref[:E],
                preferred_element_type=jnp.float32)
    h = h + jnp.dot(ie_ref[...].astype(bf), w1_ref[E:],
                    preferred_element_type=jnp.float32)
    h = h + b1_ref[...]
    h = jnp.where(h > 0, h, (jnp.exp(h) - 1.0))
    h = jnp.dot(h.astype(bf), w2_ref[...],
                preferred_element_type=jnp.float32) + b2_ref[...]
    h = jnp.where(h > 0, h, (jnp.exp(h) - 1.0))
    o = jnp.dot(h.astype(bf), w3_ref[...],
                preferred_element_type=jnp.float32) + b3_ref[...]
    out_ref[...] = o.reshape(_BT // 128, 128)


_BT = 2048


def _mlp(ue, ie, W1, b1, W2, b2, W3, b3):
    grid = (_BS // _BT,)
    full = lambda i: (0, 0)
    return pl.pallas_call(
        _mlp_body,
        grid=grid,
        in_specs=[
            pl.BlockSpec((_BT, E), lambda i: (i, 0)),
            pl.BlockSpec((_BT, E), lambda i: (i, 0)),
            pl.BlockSpec((2 * E, H1), full),
            pl.BlockSpec((1, H1), full),
            pl.BlockSpec((H1, H2), full),
            pl.BlockSpec((1, H2), full),
            pl.BlockSpec((H2, 1), full),
            pl.BlockSpec((1, 1), full),
        ],
        out_specs=pl.BlockSpec((_BT // 128, 128), lambda i: (i, 0)),
        out_shape=jax.ShapeDtypeStruct((_BS // 128, 128), jnp.float32),
        compiler_params=pltpu.CompilerParams(
            dimension_semantics=("arbitrary",),
        ),
    )(ue, ie, W1, b1, W2, b2, W3, b3)


def kernel(x, a, user_table, item_table, W1, b1, W2, b2, W3, b3):
    x2 = x.astype(jnp.int32).reshape(B // _CH, _CH)
    a2 = a.astype(jnp.int32).reshape(B // _CH, _CH)
    w1 = W1.astype(jnp.bfloat16)
    w2 = W2.astype(jnp.bfloat16)
    w3 = W3.astype(jnp.bfloat16)
    b1r = b1.reshape(1, H1)
    b2r = b2.reshape(1, H2)
    b3r = b3.reshape(1, 1)
    rows = _BS // _CH
    gather = _sc_gather_fn()
    embs = [gather(x2[k * rows:(k + 1) * rows], a2[k * rows:(k + 1) * rows],
                   user_table, item_table) for k in range(_NSPLIT)]
    outs = [_mlp(ue, ie, w1, b1r, w2, b2r, w3, b3r) for ue, ie in embs]
    return jnp.concatenate(outs, axis=0).reshape(B, 1)


# speedup vs baseline: 2.8697x; 1.0902x over previous
import functools

import jax
import jax.numpy as jnp
from jax import lax
from jax.experimental import pallas as pl
from jax.experimental.pallas import tpu as pltpu
from jax.experimental.pallas import tpu_sc as plsc

B = 16384
V = 100000
E = 128
H1 = 1024
H2 = 512

_NSPLIT = 2
_BS = B // _NSPLIT

_NC = 2
_NS = 16
_NW = _NC * _NS
_BPW = _BS // _NW
_CH = 128
_NCH = _BPW // _CH


@functools.lru_cache(maxsize=None)
def _sc_gather_fn(slice_k: int):
    mesh = plsc.VectorSubcoreMesh(core_axis_name="c", subcore_axis_name="s")
    row0 = slice_k * (_BS // _CH)

    @functools.partial(
        pl.kernel,
        mesh=mesh,
        out_type=jax.ShapeDtypeStruct((_BS, 2 * E), jnp.float32),
        scratch_types=[
            pltpu.VMEM((_NCH, _CH), jnp.int32),
            pltpu.VMEM((_NCH, _CH), jnp.int32),
            pltpu.VMEM((_CH, E), jnp.float32),
            pltpu.VMEM((_CH, E), jnp.float32),
            pltpu.SemaphoreType.DMA,
            pltpu.SemaphoreType.DMA,
            pltpu.SemaphoreType.DMA,
        ],
    )
    def _sc_gather(x_hbm, a_hbm, ut_hbm, it_hbm, emb_hbm,
                   xidx, aidx, buf0, buf1, gsem, psem0, psem1):
        wid = lax.axis_index("s") * _NC + lax.axis_index("c")
        base = wid * _BPW
        pltpu.sync_copy(x_hbm.at[pl.ds(row0 + wid * _NCH, _NCH)], xidx)
        pltpu.sync_copy(a_hbm.at[pl.ds(row0 + wid * _NCH, _NCH)], aidx)
        bufs = (buf0, buf1)
        psems = (psem0, psem1)
        puts = [None, None]
        for t in range(2 * _NCH):
            j = t % _NCH
            if t < _NCH:
                table, idxs, col = ut_hbm, xidx, 0
            else:
                table, idxs, col = it_hbm, aidx, E
            k = t % 2
            if puts[k] is not None:
                puts[k].wait()
            pltpu.async_copy(table.at[idxs.at[j]], bufs[k], gsem).wait()
            puts[k] = pltpu.async_copy(
                bufs[k],
                emb_hbm.at[pl.ds(base + j * _CH, _CH), pl.ds(col, E)],
                psems[k])
        puts[0].wait()
        puts[1].wait()

    return _sc_gather


def _mlp_body(emb_ref, w1_ref, b1_ref, w2_ref, b2_ref,
              w3_ref, b3_ref, out_ref):
    bf = jnp.bfloat16
    h = jnp.dot(emb_ref[...].astype(bf), w1_ref[...],
                preferred_element_type=jnp.float32)
    h = h + b1_ref[...]
    h = jnp.where(h > 0, h, (jnp.exp(h) - 1.0))
    h = jnp.dot(h.astype(bf), w2_ref[...],
                preferred_element_type=jnp.float32) + b2_ref[...]
    h = jnp.where(h > 0, h, (jnp.exp(h) - 1.0))
    o = jnp.dot(h.astype(bf), w3_ref[...],
                preferred_element_type=jnp.float32) + b3_ref[...]
    out_ref[...] = o.reshape(_BT // 128, 128)


_BT = 2048


def _mlp(emb, W1, b1, W2, b2, W3, b3):
    grid = (_BS // _BT,)
    full = lambda i: (0, 0)
    return pl.pallas_call(
        _mlp_body,
        grid=grid,
        in_specs=[
            pl.BlockSpec((_BT, 2 * E), lambda i: (i, 0)),
            pl.BlockSpec((2 * E, H1), full),
            pl.BlockSpec((1, H1), full),
            pl.BlockSpec((H1, H2), full),
            pl.BlockSpec((1, H2), full),
            pl.BlockSpec((H2, 1), full),
            pl.BlockSpec((1, 1), full),
        ],
        out_specs=pl.BlockSpec((_BT // 128, 128), lambda i: (i, 0)),
        out_shape=jax.ShapeDtypeStruct((_BS // 128, 128), jnp.float32),
        compiler_params=pltpu.CompilerParams(
            dimension_semantics=("arbitrary",),
        ),
    )(emb, W1, b1, W2, b2, W3, b3)


def kernel(x, a, user_table, item_table, W1, b1, W2, b2, W3, b3):
    x2 = x.astype(jnp.int32).reshape(B // _CH, _CH)
    a2 = a.astype(jnp.int32).reshape(B // _CH, _CH)
    w1 = W1.astype(jnp.bfloat16)
    w2 = W2.astype(jnp.bfloat16)
    w3 = W3.astype(jnp.bfloat16)
    b1r = b1.reshape(1, H1)
    b2r = b2.reshape(1, H2)
    b3r = b3.reshape(1, 1)
    embs = [_sc_gather_fn(k)(x2, a2, user_table, item_table)
            for k in range(_NSPLIT)]
    outs = [_mlp(emb, w1, b1r, w2, b2r, w3, b3r) for emb in embs]
    return jnp.concatenate(outs, axis=0).reshape(B, 1)


# speedup vs baseline: 2.8817x; 1.0042x over previous
import functools

import jax
import jax.numpy as jnp
from jax import lax
from jax.experimental import pallas as pl
from jax.experimental.pallas import tpu as pltpu
from jax.experimental.pallas import tpu_sc as plsc

B = 16384
V = 100000
E = 128
H1 = 1024
H2 = 512

_NSPLIT = 2
_BS = B // _NSPLIT

_NC = 2
_NS = 16
_NW = _NC * _NS
_BPW = _BS // _NW
_CH = 128
_NCH = _BPW // _CH


@functools.lru_cache(maxsize=None)
def _sc_gather_fn(slice_k: int):
    mesh = plsc.VectorSubcoreMesh(core_axis_name="c", subcore_axis_name="s")
    row0 = slice_k * (_BS // _CH)

    @functools.partial(
        pl.kernel,
        mesh=mesh,
        out_type=jax.ShapeDtypeStruct((_BS, 2 * E), jnp.float32),
        scratch_types=[
            pltpu.VMEM((_NCH, _CH), jnp.int32),
            pltpu.VMEM((_NCH, _CH), jnp.int32),
            pltpu.VMEM((_CH, E), jnp.float32),
            pltpu.VMEM((_CH, E), jnp.float32),
            pltpu.SemaphoreType.DMA,
            pltpu.SemaphoreType.DMA,
            pltpu.SemaphoreType.DMA,
        ],
    )
    def _sc_gather(x_hbm, a_hbm, ut_hbm, it_hbm, emb_hbm,
                   xidx, aidx, buf0, buf1, gsem, psem0, psem1):
        wid = lax.axis_index("s") * _NC + lax.axis_index("c")
        base = wid * _BPW
        pltpu.sync_copy(x_hbm.at[pl.ds(row0 + wid * _NCH, _NCH)], xidx)
        pltpu.sync_copy(a_hbm.at[pl.ds(row0 + wid * _NCH, _NCH)], aidx)
        bufs = (buf0, buf1)
        psems = (psem0, psem1)
        puts = [None, None]
        for t in range(2 * _NCH):
            j = t % _NCH
            if t < _NCH:
                table, idxs, col = ut_hbm, xidx, 0
            else:
                table, idxs, col = it_hbm, aidx, E
            k = t % 2
            if puts[k] is not None:
                puts[k].wait()
            pltpu.async_copy(table.at[idxs.at[j]], bufs[k], gsem).wait()
            puts[k] = pltpu.async_copy(
                bufs[k],
                emb_hbm.at[pl.ds(base + j * _CH, _CH), pl.ds(col, E)],
                psems[k])
        puts[0].wait()
        puts[1].wait()

    return _sc_gather


def _mlp_body(emb_ref, w1_ref, b1_ref, w2_ref, b2_ref,
              w3_ref, b3_ref, out_ref):
    bf = jnp.bfloat16
    h = jnp.dot(emb_ref[...].astype(bf), w1_ref[...],
                preferred_element_type=jnp.float32)
    h = h + b1_ref[...]
    h = jnp.where(h > 0, h, (jnp.exp(h) - 1.0))
    h = jnp.dot(h.astype(bf), w2_ref[...],
                preferred_element_type=jnp.float32) + b2_ref[...]
    h = jnp.where(h > 0, h, (jnp.exp(h) - 1.0))
    o = jnp.dot(h.astype(bf), w3_ref[...],
                preferred_element_type=jnp.float32) + b3_ref[...]
    out_ref[...] = o.reshape(_BT // 128, 128)


_BT = 4096


def _mlp(emb, W1, b1, W2, b2, W3, b3):
    grid = (_BS // _BT,)
    full = lambda i: (0, 0)
    return pl.pallas_call(
        _mlp_body,
        grid=grid,
        in_specs=[
            pl.BlockSpec((_BT, 2 * E), lambda i: (i, 0)),
            pl.BlockSpec((2 * E, H1), full),
            pl.BlockSpec((1, H1), full),
            pl.BlockSpec((H1, H2), full),
            pl.BlockSpec((1, H2), full),
            pl.BlockSpec((H2, 1), full),
            pl.BlockSpec((1, 1), full),
        ],
        out_specs=pl.BlockSpec((_BT // 128, 128), lambda i: (i, 0)),
        out_shape=jax.ShapeDtypeStruct((_BS // 128, 128), jnp.float32),
        compiler_params=pltpu.CompilerParams(
            dimension_semantics=("arbitrary",),
        ),
    )(emb, W1, b1, W2, b2, W3, b3)


def kernel(x, a, user_table, item_table, W1, b1, W2, b2, W3, b3):
    x2 = x.astype(jnp.int32).reshape(B // _CH, _CH)
    a2 = a.astype(jnp.int32).reshape(B // _CH, _CH)
    w1 = W1.astype(jnp.bfloat16)
    w2 = W2.astype(jnp.bfloat16)
    w3 = W3.astype(jnp.bfloat16)
    b1r = b1.reshape(1, H1)
    b2r = b2.reshape(1, H2)
    b3r = b3.reshape(1, 1)
    embs = [_sc_gather_fn(k)(x2, a2, user_table, item_table)
            for k in range(_NSPLIT)]
    outs = [_mlp(emb, w1, b1r, w2, b2r, w3, b3r) for emb in embs]
    return jnp.concatenate(outs, axis=0).reshape(B, 1)
